# baseline (speedup 1.0000x reference)
import functools

import jax
import jax.numpy as jnp
from jax import lax
from jax.experimental import pallas as pl
from jax.experimental.pallas import tpu as pltpu
from jax.experimental.pallas import tpu_sc as plsc

BATCH = 16384
DIM = 64
PACKED_DIM = 2 * DIM
LANES = 16
NUM_CORES = 2
NUM_SUBCORES = 16
NUM_WORKERS = NUM_CORES * NUM_SUBCORES
ROWS_PER_WORKER = BATCH // NUM_WORKERS
CHUNK = 256
NCHUNKS = ROWS_PER_WORKER // CHUNK
GROUPS = CHUNK // LANES

_mesh = plsc.VectorSubcoreMesh(core_axis_name="c", subcore_axis_name="s")

_cp = pltpu.CompilerParams(needs_layout_passes=False,
                           use_tc_tiling_on_sc=True)


@functools.partial(
    pl.kernel,
    out_type=jax.ShapeDtypeStruct((BATCH,), jnp.float32),
    mesh=_mesh,
    compiler_params=_cp,
    scratch_types=[
        pltpu.VMEM((CHUNK,), jnp.int32),
        pltpu.VMEM((CHUNK,), jnp.int32),
        pltpu.VMEM((CHUNK,), jnp.int32),
        pltpu.VMEM((CHUNK,), jnp.int32),
        pltpu.VMEM((CHUNK,), jnp.int32),
        pltpu.VMEM((CHUNK,), jnp.int32),
        pltpu.VMEM((CHUNK, PACKED_DIM), jnp.float32),
        pltpu.VMEM((CHUNK, PACKED_DIM), jnp.float32),
        pltpu.VMEM((CHUNK, PACKED_DIM), jnp.float32),
        pltpu.VMEM((CHUNK,), jnp.float32),
        pltpu.SemaphoreType.DMA,
        pltpu.SemaphoreType.DMA,
        pltpu.SemaphoreType.DMA,
    ],
)
def _transe_sc(hg_hbm, rg_hbm, tg_hbm, hp_hbm, rp_hbm, tp_hbm,
               ent_hbm, rel_hbm, out_hbm,
               hg_v, rg_v, tg_v, hp_v, rp_v, tp_v,
               h_v, r_v, t_v, o_v, sem_h, sem_r, sem_t):
    wid = lax.axis_index("s") * NUM_CORES + lax.axis_index("c")

    @pl.loop(0, NCHUNKS)
    def _chunk(c):
        base = wid * ROWS_PER_WORKER + c * CHUNK

        pltpu.sync_copy(hg_hbm.at[pl.ds(base, CHUNK)], hg_v)
        pltpu.sync_copy(rg_hbm.at[pl.ds(base, CHUNK)], rg_v)
        pltpu.sync_copy(tg_hbm.at[pl.ds(base, CHUNK)], tg_v)
        pltpu.sync_copy(hp_hbm.at[pl.ds(base, CHUNK)], hp_v)
        pltpu.sync_copy(rp_hbm.at[pl.ds(base, CHUNK)], rp_v)
        pltpu.sync_copy(tp_hbm.at[pl.ds(base, CHUNK)], tp_v)

        ch = pltpu.async_copy(ent_hbm.at[hg_v], h_v, sem_h)
        cr = pltpu.async_copy(rel_hbm.at[rg_v], r_v, sem_r)
        ct = pltpu.async_copy(ent_hbm.at[tg_v], t_v, sem_t)
        ch.wait()
        cr.wait()
        ct.wait()

        @pl.loop(0, GROUPS)
        def _group(g):
            rows = g * LANES + lax.iota(jnp.int32, LANES)
            hp = hp_v[pl.ds(g * LANES, LANES)]
            rp = rp_v[pl.ds(g * LANES, LANES)]
            tp = tp_v[pl.ds(g * LANES, LANES)]

            def body(d, acc):
                h = plsc.load_gather(h_v, [rows, hp + d])
                r = plsc.load_gather(r_v, [rows, rp + d])
                t = plsc.load_gather(t_v, [rows, tp + d])
                return acc + jnp.abs(h + r - t)

            acc = lax.fori_loop(0, DIM, body, jnp.zeros((LANES,), jnp.float32))
            o_v[pl.ds(g * LANES, LANES)] = acc

        pltpu.sync_copy(o_v, out_hbm.at[pl.ds(base, CHUNK)])


_CONV_ENTS = 1024
_CONV_GRID = -(-1000000 // _CONV_ENTS)


def _conv_body(et_ref, out_ref):
    x = et_ref[...].astype(jnp.bfloat16)
    eye = jnp.eye(DIM, dtype=jnp.bfloat16)
    y = jax.lax.dot_general(x, eye, (((0,), (0,)), ((), ())),
                            preferred_element_type=jnp.float32)
    half = _CONV_ENTS // 2
    out_ref[:, 0:DIM] = y[0:half, :]
    out_ref[:, DIM:PACKED_DIM] = y[half:_CONV_ENTS, :]


_convert = pl.pallas_call(
    _conv_body,
    grid=(_CONV_GRID,),
    in_specs=[pl.BlockSpec((DIM, _CONV_ENTS), lambda j: (0, j))],
    out_specs=pl.BlockSpec((_CONV_ENTS // 2, PACKED_DIM), lambda j: (j, 0)),
    out_shape=jax.ShapeDtypeStruct((_CONV_GRID * _CONV_ENTS // 2, PACKED_DIM),
                                   jnp.float32),
)


def kernel(sample, entity_embedding, relation_embedding):
    idx = sample.astype(jnp.int32)
    packed = ((idx >> 10) << 9) | (idx & 511)
    half = ((idx >> 9) & 1) << 6
    ent2 = _convert(entity_embedding.T)
    rel2 = _convert(relation_embedding.T)
    scores = _transe_sc(packed[:, 0], packed[:, 1], packed[:, 2],
                        half[:, 0], half[:, 1], half[:, 2],
                        ent2, rel2)
    return scores.reshape(BATCH, 1)


# speedup vs baseline: 3.2202x; 3.2202x over previous
import functools

import jax
import jax.numpy as jnp
from jax import lax
from jax.experimental import pallas as pl
from jax.experimental.pallas import tpu as pltpu
from jax.experimental.pallas import tpu_sc as plsc

BATCH = 16384
DIM = 64
PACKED_DIM = 2 * DIM
LANES = 16
NUM_CORES = 2
NUM_SUBCORES = 16
NUM_WORKERS = NUM_CORES * NUM_SUBCORES
ROWS_PER_WORKER = BATCH // NUM_WORKERS
CHUNK = 256
NCHUNKS = ROWS_PER_WORKER // CHUNK
GROUPS = CHUNK // LANES

_mesh = plsc.VectorSubcoreMesh(core_axis_name="c", subcore_axis_name="s")

_cp = pltpu.CompilerParams(needs_layout_passes=False,
                           use_tc_tiling_on_sc=True)


@functools.partial(
    pl.kernel,
    out_type=jax.ShapeDtypeStruct((BATCH,), jnp.float32),
    mesh=_mesh,
    compiler_params=_cp,
    scratch_types=[
        pltpu.VMEM((CHUNK,), jnp.int32),
        pltpu.VMEM((CHUNK,), jnp.int32),
        pltpu.VMEM((CHUNK,), jnp.int32),
        pltpu.VMEM((CHUNK,), jnp.int32),
        pltpu.VMEM((CHUNK,), jnp.int32),
        pltpu.VMEM((CHUNK,), jnp.int32),
        pltpu.VMEM((CHUNK, PACKED_DIM), jnp.float32),
        pltpu.VMEM((CHUNK, PACKED_DIM), jnp.float32),
        pltpu.VMEM((CHUNK, PACKED_DIM), jnp.float32),
        pltpu.VMEM((CHUNK,), jnp.float32),
        pltpu.SemaphoreType.DMA,
        pltpu.SemaphoreType.DMA,
        pltpu.SemaphoreType.DMA,
    ],
)
def _transe_sc(hg_hbm, rg_hbm, tg_hbm, hp_hbm, rp_hbm, tp_hbm,
               ent_hbm, rel_hbm, out_hbm,
               hg_v, rg_v, tg_v, hp_v, rp_v, tp_v,
               h_v, r_v, t_v, o_v, sem_h, sem_r, sem_t):
    wid = lax.axis_index("s") * NUM_CORES + lax.axis_index("c")

    @pl.loop(0, NCHUNKS)
    def _chunk(c):
        base = wid * ROWS_PER_WORKER + c * CHUNK

        pltpu.sync_copy(hg_hbm.at[pl.ds(base, CHUNK)], hg_v)
        pltpu.sync_copy(rg_hbm.at[pl.ds(base, CHUNK)], rg_v)
        pltpu.sync_copy(tg_hbm.at[pl.ds(base, CHUNK)], tg_v)
        pltpu.sync_copy(hp_hbm.at[pl.ds(base, CHUNK)], hp_v)
        pltpu.sync_copy(rp_hbm.at[pl.ds(base, CHUNK)], rp_v)
        pltpu.sync_copy(tp_hbm.at[pl.ds(base, CHUNK)], tp_v)

        ch = pltpu.async_copy(ent_hbm.at[hg_v], h_v, sem_h)
        cr = pltpu.async_copy(rel_hbm.at[rg_v], r_v, sem_r)
        ct = pltpu.async_copy(ent_hbm.at[tg_v], t_v, sem_t)
        ch.wait()
        cr.wait()
        ct.wait()

        @pl.loop(0, GROUPS)
        def _group(g):
            rows = g * LANES + lax.iota(jnp.int32, LANES)
            hp = hp_v[pl.ds(g * LANES, LANES)]
            rp = rp_v[pl.ds(g * LANES, LANES)]
            tp = tp_v[pl.ds(g * LANES, LANES)]

            def body(d, acc):
                h = plsc.load_gather(h_v, [rows, hp + d])
                r = plsc.load_gather(r_v, [rows, rp + d])
                t = plsc.load_gather(t_v, [rows, tp + d])
                return acc + jnp.abs(h + r - t)

            acc = lax.fori_loop(0, DIM, body, jnp.zeros((LANES,), jnp.float32))
            o_v[pl.ds(g * LANES, LANES)] = acc

        pltpu.sync_copy(o_v, out_hbm.at[pl.ds(base, CHUNK)])


_CONV_ENTS = 8192
_CONV_HALF = _CONV_ENTS // 2
_CONV_GRID = -(-1000000 // _CONV_ENTS)


def _conv_body(et_ref, out_ref):
    x = et_ref[...].astype(jnp.bfloat16)
    xp = jnp.concatenate([x[:, :_CONV_HALF], x[:, _CONV_HALF:]], axis=0)
    eye = jnp.eye(PACKED_DIM, dtype=jnp.bfloat16)
    out_ref[...] = jax.lax.dot_general(xp, eye, (((0,), (0,)), ((), ())),
                                       preferred_element_type=jnp.float32)


_convert = pl.pallas_call(
    _conv_body,
    grid=(_CONV_GRID,),
    in_specs=[pl.BlockSpec((DIM, _CONV_ENTS), lambda j: (0, j))],
    out_specs=pl.BlockSpec((_CONV_HALF, PACKED_DIM), lambda j: (j, 0)),
    out_shape=jax.ShapeDtypeStruct((_CONV_GRID * _CONV_HALF, PACKED_DIM),
                                   jnp.float32),
)


def kernel(sample, entity_embedding, relation_embedding):
    idx = sample.astype(jnp.int32)
    packed = ((idx >> 13) << 12) | (idx & (_CONV_HALF - 1))
    half = ((idx >> 12) & 1) << 6
    ent2 = _convert(entity_embedding.T)
    rel2 = _convert(relation_embedding.T)
    scores = _transe_sc(packed[:, 0], packed[:, 1], packed[:, 2],
                        half[:, 0], half[:, 1], half[:, 2],
                        ent2, rel2)
    return scores.reshape(BATCH, 1)


# speedup vs baseline: 3.7468x; 1.1635x over previous
import functools

import jax
import jax.numpy as jnp
from jax import lax
from jax.experimental import pallas as pl
from jax.experimental.pallas import tpu as pltpu
from jax.experimental.pallas import tpu_sc as plsc

BATCH = 16384
DIM = 64
PACKED_DIM = 2 * DIM
LANES = 16
NUM_CORES = 2
NUM_SUBCORES = 16
NUM_WORKERS = NUM_CORES * NUM_SUBCORES
ROWS_PER_WORKER = BATCH // NUM_WORKERS
CHUNK = 256
NCHUNKS = ROWS_PER_WORKER // CHUNK
GROUPS = CHUNK // LANES

_mesh = plsc.VectorSubcoreMesh(core_axis_name="c", subcore_axis_name="s")

_cp = pltpu.CompilerParams(needs_layout_passes=False,
                           use_tc_tiling_on_sc=True)


@functools.partial(
    pl.kernel,
    out_type=jax.ShapeDtypeStruct((BATCH,), jnp.float32),
    mesh=_mesh,
    compiler_params=_cp,
    scratch_types=[
        pltpu.VMEM((CHUNK,), jnp.int32),
        pltpu.VMEM((CHUNK,), jnp.int32),
        pltpu.VMEM((CHUNK,), jnp.int32),
        pltpu.VMEM((CHUNK,), jnp.int32),
        pltpu.VMEM((CHUNK,), jnp.int32),
        pltpu.VMEM((CHUNK,), jnp.int32),
        pltpu.VMEM((CHUNK, PACKED_DIM), jnp.int32),
        pltpu.VMEM((CHUNK, PACKED_DIM), jnp.int32),
        pltpu.VMEM((CHUNK, PACKED_DIM), jnp.int32),
        pltpu.VMEM((CHUNK,), jnp.float32),
        pltpu.SemaphoreType.DMA,
        pltpu.SemaphoreType.DMA,
        pltpu.SemaphoreType.DMA,
    ],
)
def _transe_sc(hg_hbm, rg_hbm, tg_hbm, hp_hbm, rp_hbm, tp_hbm,
               ent_hbm, rel_hbm, out_hbm,
               hg_v, rg_v, tg_v, hp_v, rp_v, tp_v,
               h_v, r_v, t_v, o_v, sem_h, sem_r, sem_t):
    wid = lax.axis_index("s") * NUM_CORES + lax.axis_index("c")

    @pl.loop(0, NCHUNKS)
    def _chunk(c):
        base = wid * ROWS_PER_WORKER + c * CHUNK

        pltpu.sync_copy(hg_hbm.at[pl.ds(base, CHUNK)], hg_v)
        pltpu.sync_copy(rg_hbm.at[pl.ds(base, CHUNK)], rg_v)
        pltpu.sync_copy(tg_hbm.at[pl.ds(base, CHUNK)], tg_v)
        pltpu.sync_copy(hp_hbm.at[pl.ds(base, CHUNK)], hp_v)
        pltpu.sync_copy(rp_hbm.at[pl.ds(base, CHUNK)], rp_v)
        pltpu.sync_copy(tp_hbm.at[pl.ds(base, CHUNK)], tp_v)

        ch = pltpu.async_copy(ent_hbm.at[hg_v], h_v, sem_h)
        cr = pltpu.async_copy(rel_hbm.at[rg_v], r_v, sem_r)
        ct = pltpu.async_copy(ent_hbm.at[tg_v], t_v, sem_t)
        ch.wait()
        cr.wait()
        ct.wait()

        @pl.loop(0, GROUPS)
        def _group(g):
            rows = g * LANES + lax.iota(jnp.int32, LANES)
            hp = hp_v[pl.ds(g * LANES, LANES)]
            rp = rp_v[pl.ds(g * LANES, LANES)]
            tp = tp_v[pl.ds(g * LANES, LANES)]
            himask = jnp.full((LANES,), -0x10000, jnp.int32)

            def unpack(v):
                lo = lax.bitcast_convert_type(lax.shift_left(v, 16),
                                              jnp.float32)
                hi = lax.bitcast_convert_type(v & himask, jnp.float32)
                return lo, hi

            def body(d, accs):
                acc_lo, acc_hi = accs
                hl, hh = unpack(plsc.load_gather(h_v, [rows, hp + d]))
                rl, rh = unpack(plsc.load_gather(r_v, [rows, rp + d]))
                tl, th = unpack(plsc.load_gather(t_v, [rows, tp + d]))
                return (acc_lo + jnp.abs(hl + rl - tl),
                        acc_hi + jnp.abs(hh + rh - th))

            zero = jnp.zeros((LANES,), jnp.float32)
            acc_lo, acc_hi = lax.fori_loop(0, DIM // 2, body, (zero, zero))
            o_v[pl.ds(g * LANES, LANES)] = acc_lo + acc_hi

        pltpu.sync_copy(o_v, out_hbm.at[pl.ds(base, CHUNK)])


_CONV_ENTS = 8192
_CONV_Q = _CONV_ENTS // 4
_CONV_GRID = -(-1000000 // _CONV_ENTS)


def _conv_body(et_ref, out_ref):
    x = et_ref[...].astype(jnp.bfloat16)
    xp = jnp.concatenate([x[:, 0 * _CONV_Q:1 * _CONV_Q],
                          x[:, 1 * _CONV_Q:2 * _CONV_Q],
                          x[:, 2 * _CONV_Q:3 * _CONV_Q],
                          x[:, 3 * _CONV_Q:4 * _CONV_Q]], axis=0)
    kk = lax.broadcasted_iota(jnp.int32, (4 * DIM, 4 * DIM), 0)
    cc = lax.broadcasted_iota(jnp.int32, (4 * DIM, 4 * DIM), 1)
    cm = cc & 127
    ktgt = ((cm >> 5) << 6) + ((cm & 31) << 1) + (cc >> 7)
    eye_p = (kk == ktgt).astype(jnp.bfloat16)
    z = jax.lax.dot_general(xp, eye_p, (((0,), (0,)), ((), ())),
                            preferred_element_type=jnp.float32)
    zi = jax.lax.bitcast_convert_type(z, jnp.int32)
    lo = jax.lax.shift_right_logical(zi[:, 0:PACKED_DIM], 16)
    out_ref[...] = zi[:, PACKED_DIM:2 * PACKED_DIM] | lo


_convert = pl.pallas_call(
    _conv_body,
    grid=(_CONV_GRID,),
    in_specs=[pl.BlockSpec((DIM, _CONV_ENTS), lambda j: (0, j))],
    out_specs=pl.BlockSpec((_CONV_Q, PACKED_DIM), lambda j: (j, 0)),
    out_shape=jax.ShapeDtypeStruct((_CONV_GRID * _CONV_Q, PACKED_DIM),
                                   jnp.int32),
)


def kernel(sample, entity_embedding, relation_embedding):
    idx = sample.astype(jnp.int32)
    packed = ((idx >> 13) << 11) | (idx & (_CONV_Q - 1))
    half = ((idx >> 11) & 3) << 5
    ent2 = _convert(entity_embedding.T)
    rel2 = _convert(relation_embedding.T)
    scores = _transe_sc(packed[:, 0], packed[:, 1], packed[:, 2],
                        half[:, 0], half[:, 1], half[:, 2],
                        ent2, rel2)
    return scores.reshape(BATCH, 1)


# speedup vs baseline: 4.6419x; 1.2389x over previous
import functools

import jax
import jax.numpy as jnp
from jax import lax
from jax.experimental import pallas as pl
from jax.experimental.pallas import tpu as pltpu
from jax.experimental.pallas import tpu_sc as plsc

BATCH = 16384
DIM = 64
PACKED_DIM = 2 * DIM
LANES = 16
NUM_CORES = 2
NUM_SUBCORES = 16
NUM_WORKERS = NUM_CORES * NUM_SUBCORES
ROWS_PER_WORKER = BATCH // NUM_WORKERS
CHUNK = 256
NCHUNKS = ROWS_PER_WORKER // CHUNK
GROUPS = CHUNK // LANES

_mesh = plsc.VectorSubcoreMesh(core_axis_name="c", subcore_axis_name="s")

_cp = pltpu.CompilerParams(needs_layout_passes=False,
                           use_tc_tiling_on_sc=True)


@functools.partial(
    pl.kernel,
    out_type=jax.ShapeDtypeStruct((BATCH,), jnp.float32),
    mesh=_mesh,
    compiler_params=_cp,
    scratch_types=[
        pltpu.VMEM((CHUNK,), jnp.int32),
        pltpu.VMEM((CHUNK,), jnp.int32),
        pltpu.VMEM((CHUNK,), jnp.int32),
        pltpu.VMEM((CHUNK,), jnp.int32),
        pltpu.VMEM((CHUNK,), jnp.int32),
        pltpu.VMEM((CHUNK,), jnp.int32),
        pltpu.VMEM((CHUNK, PACKED_DIM), jnp.int32),
        pltpu.VMEM((CHUNK, PACKED_DIM), jnp.int32),
        pltpu.VMEM((CHUNK, PACKED_DIM), jnp.int32),
        pltpu.VMEM((CHUNK,), jnp.float32),
        pltpu.SemaphoreType.DMA,
        pltpu.SemaphoreType.DMA,
        pltpu.SemaphoreType.DMA,
    ],
)
def _transe_sc(hg_hbm, rg_hbm, tg_hbm, hp_hbm, rp_hbm, tp_hbm,
               ent_hbm, rel_hbm, out_hbm,
               hg_v, rg_v, tg_v, hp_v, rp_v, tp_v,
               h_v, r_v, t_v, o_v, sem_h, sem_r, sem_t):
    wid = lax.axis_index("s") * NUM_CORES + lax.axis_index("c")

    @pl.loop(0, NCHUNKS)
    def _chunk(c):
        base = wid * ROWS_PER_WORKER + c * CHUNK

        pltpu.sync_copy(hg_hbm.at[pl.ds(base, CHUNK)], hg_v)
        pltpu.sync_copy(rg_hbm.at[pl.ds(base, CHUNK)], rg_v)
        pltpu.sync_copy(tg_hbm.at[pl.ds(base, CHUNK)], tg_v)
        pltpu.sync_copy(hp_hbm.at[pl.ds(base, CHUNK)], hp_v)
        pltpu.sync_copy(rp_hbm.at[pl.ds(base, CHUNK)], rp_v)
        pltpu.sync_copy(tp_hbm.at[pl.ds(base, CHUNK)], tp_v)

        ch = pltpu.async_copy(ent_hbm.at[hg_v], h_v, sem_h)
        cr = pltpu.async_copy(rel_hbm.at[rg_v], r_v, sem_r)
        ct = pltpu.async_copy(ent_hbm.at[tg_v], t_v, sem_t)
        ch.wait()
        cr.wait()
        ct.wait()

        @pl.loop(0, GROUPS)
        def _group(g):
            rows = g * LANES + lax.iota(jnp.int32, LANES)
            hp = hp_v[pl.ds(g * LANES, LANES)]
            rp = rp_v[pl.ds(g * LANES, LANES)]
            tp = tp_v[pl.ds(g * LANES, LANES)]
            himask = jnp.full((LANES,), -0x10000, jnp.int32)

            def unpack(v):
                lo = lax.bitcast_convert_type(lax.shift_left(v, 16),
                                              jnp.float32)
                hi = lax.bitcast_convert_type(v & himask, jnp.float32)
                return lo, hi

            def body(d, accs):
                acc_lo, acc_hi = accs
                hl, hh = unpack(plsc.load_gather(h_v, [rows, hp + d]))
                rl, rh = unpack(plsc.load_gather(r_v, [rows, rp + d]))
                tl, th = unpack(plsc.load_gather(t_v, [rows, tp + d]))
                return (acc_lo + jnp.abs(hl + rl - tl),
                        acc_hi + jnp.abs(hh + rh - th))

            zero = jnp.zeros((LANES,), jnp.float32)
            acc_lo, acc_hi = lax.fori_loop(0, DIM // 2, body, (zero, zero))
            o_v[pl.ds(g * LANES, LANES)] = acc_lo + acc_hi

        pltpu.sync_copy(o_v, out_hbm.at[pl.ds(base, CHUNK)])


_CONV_ENTS = 16384
_CONV_Q = _CONV_ENTS // 4
_CONV_GRID = -(-1000000 // _CONV_ENTS)


def _conv_body(et_ref, out_ref):
    x = et_ref[...].astype(jnp.bfloat16)
    xp = jnp.concatenate([x[:, 0 * _CONV_Q:1 * _CONV_Q],
                          x[:, 1 * _CONV_Q:2 * _CONV_Q],
                          x[:, 2 * _CONV_Q:3 * _CONV_Q],
                          x[:, 3 * _CONV_Q:4 * _CONV_Q]], axis=0)
    kk = lax.broadcasted_iota(jnp.int32, (4 * DIM, 4 * DIM), 0)
    cc = lax.broadcasted_iota(jnp.int32, (4 * DIM, 4 * DIM), 1)
    cm = cc & 127
    ktgt = ((cm >> 5) << 6) + ((cm & 31) << 1) + (cc >> 7)
    eye_p = (kk == ktgt).astype(jnp.bfloat16)
    z = jax.lax.dot_general(xp, eye_p, (((0,), (0,)), ((), ())),
                            preferred_element_type=jnp.float32)
    zi = jax.lax.bitcast_convert_type(z, jnp.int32)
    lo = jax.lax.shift_right_logical(zi[:, 0:PACKED_DIM], 16)
    out_ref[...] = zi[:, PACKED_DIM:2 * PACKED_DIM] | lo


_convert = pl.pallas_call(
    _conv_body,
    grid=(_CONV_GRID,),
    in_specs=[pl.BlockSpec((DIM, _CONV_ENTS), lambda j: (0, j))],
    out_specs=pl.BlockSpec((_CONV_Q, PACKED_DIM), lambda j: (j, 0)),
    out_shape=jax.ShapeDtypeStruct((_CONV_GRID * _CONV_Q, PACKED_DIM),
                                   jnp.int32),
)


def kernel(sample, entity_embedding, relation_embedding):
    idx = sample.astype(jnp.int32)
    packed = ((idx >> 14) << 12) | (idx & (_CONV_Q - 1))
    half = ((idx >> 12) & 3) << 5
    ent2 = _convert(entity_embedding.T)
    rel2 = _convert(relation_embedding.T)
    scores = _transe_sc(packed[:, 0], packed[:, 1], packed[:, 2],
                        half[:, 0], half[:, 1], half[:, 2],
                        ent2, rel2)
    return scores.reshape(BATCH, 1)


# speedup vs baseline: 4.8701x; 1.0492x over previous
import functools

import jax
import jax.numpy as jnp
from jax import lax
from jax.experimental import pallas as pl
from jax.experimental.pallas import tpu as pltpu
from jax.experimental.pallas import tpu_sc as plsc

BATCH = 16384
DIM = 64
PACKED_DIM = 2 * DIM
LANES = 16
NUM_CORES = 2
NUM_SUBCORES = 16
NUM_WORKERS = NUM_CORES * NUM_SUBCORES
ROWS_PER_WORKER = BATCH // NUM_WORKERS
CHUNK = 256
NCHUNKS = ROWS_PER_WORKER // CHUNK
GROUPS = CHUNK // LANES

_mesh = plsc.VectorSubcoreMesh(core_axis_name="c", subcore_axis_name="s")

_cp = pltpu.CompilerParams(needs_layout_passes=False,
                           use_tc_tiling_on_sc=True)


@functools.partial(
    pl.kernel,
    out_type=jax.ShapeDtypeStruct((BATCH,), jnp.float32),
    mesh=_mesh,
    compiler_params=_cp,
    scratch_types=[
        pltpu.VMEM((CHUNK,), jnp.int32),
        pltpu.VMEM((CHUNK,), jnp.int32),
        pltpu.VMEM((CHUNK,), jnp.int32),
        pltpu.VMEM((CHUNK,), jnp.int32),
        pltpu.VMEM((CHUNK,), jnp.int32),
        pltpu.VMEM((CHUNK,), jnp.int32),
        pltpu.VMEM((CHUNK, PACKED_DIM), jnp.int32),
        pltpu.VMEM((CHUNK, PACKED_DIM), jnp.int32),
        pltpu.VMEM((CHUNK, PACKED_DIM), jnp.int32),
        pltpu.VMEM((CHUNK,), jnp.float32),
        pltpu.SemaphoreType.DMA,
        pltpu.SemaphoreType.DMA,
        pltpu.SemaphoreType.DMA,
    ],
)
def _transe_sc(hg_hbm, rg_hbm, tg_hbm, hp_hbm, rp_hbm, tp_hbm,
               ent_hbm, rel_hbm, out_hbm,
               hg_v, rg_v, tg_v, hp_v, rp_v, tp_v,
               h_v, r_v, t_v, o_v, sem_h, sem_r, sem_t):
    wid = lax.axis_index("s") * NUM_CORES + lax.axis_index("c")

    @pl.loop(0, NCHUNKS)
    def _chunk(c):
        base = wid * ROWS_PER_WORKER + c * CHUNK

        pltpu.sync_copy(hg_hbm.at[pl.ds(base, CHUNK)], hg_v)
        pltpu.sync_copy(rg_hbm.at[pl.ds(base, CHUNK)], rg_v)
        pltpu.sync_copy(tg_hbm.at[pl.ds(base, CHUNK)], tg_v)
        pltpu.sync_copy(hp_hbm.at[pl.ds(base, CHUNK)], hp_v)
        pltpu.sync_copy(rp_hbm.at[pl.ds(base, CHUNK)], rp_v)
        pltpu.sync_copy(tp_hbm.at[pl.ds(base, CHUNK)], tp_v)

        ch = pltpu.async_copy(ent_hbm.at[hg_v], h_v, sem_h)
        cr = pltpu.async_copy(rel_hbm.at[rg_v], r_v, sem_r)
        ct = pltpu.async_copy(ent_hbm.at[tg_v], t_v, sem_t)
        ch.wait()
        cr.wait()
        ct.wait()

        @pl.loop(0, GROUPS)
        def _group(g):
            rows = g * LANES + lax.iota(jnp.int32, LANES)
            hp = hp_v[pl.ds(g * LANES, LANES)]
            rp = rp_v[pl.ds(g * LANES, LANES)]
            tp = tp_v[pl.ds(g * LANES, LANES)]
            himask = jnp.full((LANES,), -0x10000, jnp.int32)

            def unpack(v):
                lo = lax.bitcast_convert_type(lax.shift_left(v, 16),
                                              jnp.float32)
                hi = lax.bitcast_convert_type(v & himask, jnp.float32)
                return lo, hi

            def body(d, accs):
                acc_lo, acc_hi = accs
                hl, hh = unpack(plsc.load_gather(h_v, [rows, hp + d]))
                rl, rh = unpack(plsc.load_gather(r_v, [rows, rp + d]))
                tl, th = unpack(plsc.load_gather(t_v, [rows, tp + d]))
                return (acc_lo + jnp.abs(hl + rl - tl),
                        acc_hi + jnp.abs(hh + rh - th))

            zero = jnp.zeros((LANES,), jnp.float32)
            acc_lo, acc_hi = lax.fori_loop(0, DIM // 2, body, (zero, zero))
            o_v[pl.ds(g * LANES, LANES)] = acc_lo + acc_hi

        pltpu.sync_copy(o_v, out_hbm.at[pl.ds(base, CHUNK)])


_CONV_ENTS = 32768
_CONV_Q = _CONV_ENTS // 4
_CONV_GRID = -(-1000000 // _CONV_ENTS)


def _conv_body(et_ref, out_ref):
    x = et_ref[...].astype(jnp.bfloat16)
    xp = jnp.concatenate([x[:, 0 * _CONV_Q:1 * _CONV_Q],
                          x[:, 1 * _CONV_Q:2 * _CONV_Q],
                          x[:, 2 * _CONV_Q:3 * _CONV_Q],
                          x[:, 3 * _CONV_Q:4 * _CONV_Q]], axis=0)
    kk = lax.broadcasted_iota(jnp.int32, (4 * DIM, 4 * DIM), 0)
    cc = lax.broadcasted_iota(jnp.int32, (4 * DIM, 4 * DIM), 1)
    cm = cc & 127
    ktgt = ((cm >> 5) << 6) + ((cm & 31) << 1) + (cc >> 7)
    eye_p = (kk == ktgt).astype(jnp.bfloat16)
    z = jax.lax.dot_general(xp, eye_p, (((0,), (0,)), ((), ())),
                            preferred_element_type=jnp.float32)
    zi = jax.lax.bitcast_convert_type(z, jnp.int32)
    lo = jax.lax.shift_right_logical(zi[:, 0:PACKED_DIM], 16)
    out_ref[...] = zi[:, PACKED_DIM:2 * PACKED_DIM] | lo


_convert = pl.pallas_call(
    _conv_body,
    grid=(_CONV_GRID,),
    in_specs=[pl.BlockSpec((DIM, _CONV_ENTS), lambda j: (0, j))],
    out_specs=pl.BlockSpec((_CONV_Q, PACKED_DIM), lambda j: (j, 0)),
    out_shape=jax.ShapeDtypeStruct((_CONV_GRID * _CONV_Q, PACKED_DIM),
                                   jnp.int32),
)


def kernel(sample, entity_embedding, relation_embedding):
    idx = sample.astype(jnp.int32)
    packed = ((idx >> 15) << 13) | (idx & (_CONV_Q - 1))
    half = ((idx >> 13) & 3) << 5
    ent2 = _convert(entity_embedding.T)
    rel2 = _convert(relation_embedding.T)
    scores = _transe_sc(packed[:, 0], packed[:, 1], packed[:, 2],
                        half[:, 0], half[:, 1], half[:, 2],
                        ent2, rel2)
    return scores.reshape(BATCH, 1)


# speedup vs baseline: 5.0022x; 1.0271x over previous
import functools

import jax
import jax.numpy as jnp
from jax import lax
from jax.experimental import pallas as pl
from jax.experimental.pallas import tpu as pltpu
from jax.experimental.pallas import tpu_sc as plsc

BATCH = 16384
DIM = 64
PACKED_DIM = 2 * DIM
LANES = 16
NUM_CORES = 2
NUM_SUBCORES = 16
NUM_WORKERS = NUM_CORES * NUM_SUBCORES
ROWS_PER_WORKER = BATCH // NUM_WORKERS
CHUNK = 256
NCHUNKS = ROWS_PER_WORKER // CHUNK
GROUPS = CHUNK // LANES

_mesh = plsc.VectorSubcoreMesh(core_axis_name="c", subcore_axis_name="s")

_cp = pltpu.CompilerParams(needs_layout_passes=False,
                           use_tc_tiling_on_sc=True)


@functools.partial(
    pl.kernel,
    out_type=jax.ShapeDtypeStruct((BATCH,), jnp.float32),
    mesh=_mesh,
    compiler_params=_cp,
    scratch_types=[
        pltpu.VMEM((8, CHUNK), jnp.int32),
        pltpu.VMEM((CHUNK,), jnp.int32),
        pltpu.VMEM((CHUNK,), jnp.int32),
        pltpu.VMEM((CHUNK,), jnp.int32),
        pltpu.VMEM((CHUNK, PACKED_DIM), jnp.int32),
        pltpu.VMEM((CHUNK, PACKED_DIM), jnp.int32),
        pltpu.VMEM((CHUNK, PACKED_DIM), jnp.int32),
        pltpu.VMEM((CHUNK,), jnp.float32),
        pltpu.SemaphoreType.DMA,
        pltpu.SemaphoreType.DMA,
        pltpu.SemaphoreType.DMA,
    ],
)
def _transe_sc(idx_hbm, ent_hbm, rel_hbm, out_hbm,
               idx_v, hg_v, rg_v, tg_v, h_v, r_v, t_v, o_v,
               sem_h, sem_r, sem_t):
    wid = lax.axis_index("s") * NUM_CORES + lax.axis_index("c")

    @pl.loop(0, NCHUNKS)
    def _chunk(c):
        base = wid * ROWS_PER_WORKER + c * CHUNK

        pltpu.sync_copy(idx_hbm.at[:, pl.ds(base, CHUNK)], idx_v)

        @pl.loop(0, CHUNK // LANES)
        def _stage(i):
            sl = pl.ds(i * LANES, LANES)
            hg_v[sl] = idx_v[0, sl]
            rg_v[sl] = idx_v[1, sl]
            tg_v[sl] = idx_v[2, sl]

        ch = pltpu.async_copy(ent_hbm.at[hg_v], h_v, sem_h)
        cr = pltpu.async_copy(rel_hbm.at[rg_v], r_v, sem_r)
        ct = pltpu.async_copy(ent_hbm.at[tg_v], t_v, sem_t)
        ch.wait()
        cr.wait()
        ct.wait()

        @pl.loop(0, GROUPS)
        def _group(g):
            rows = g * LANES + lax.iota(jnp.int32, LANES)
            hp = idx_v[3, pl.ds(g * LANES, LANES)]
            rp = idx_v[4, pl.ds(g * LANES, LANES)]
            tp = idx_v[5, pl.ds(g * LANES, LANES)]
            himask = jnp.full((LANES,), -0x10000, jnp.int32)

            def unpack(v):
                lo = lax.bitcast_convert_type(lax.shift_left(v, 16),
                                              jnp.float32)
                hi = lax.bitcast_convert_type(v & himask, jnp.float32)
                return lo, hi

            def body(d, accs):
                acc_lo, acc_hi = accs
                hl, hh = unpack(plsc.load_gather(h_v, [rows, hp + d]))
                rl, rh = unpack(plsc.load_gather(r_v, [rows, rp + d]))
                tl, th = unpack(plsc.load_gather(t_v, [rows, tp + d]))
                return (acc_lo + jnp.abs(hl + rl - tl),
                        acc_hi + jnp.abs(hh + rh - th))

            zero = jnp.zeros((LANES,), jnp.float32)
            acc_lo, acc_hi = lax.fori_loop(0, DIM // 2, body, (zero, zero),
                                           unroll=8)
            o_v[pl.ds(g * LANES, LANES)] = acc_lo + acc_hi

        pltpu.sync_copy(o_v, out_hbm.at[pl.ds(base, CHUNK)])


_CONV_ENTS = 32768
_CONV_Q = _CONV_ENTS // 4
_CONV_GRID = -(-1000000 // _CONV_ENTS)


def _conv_body(et_ref, out_ref):
    x = et_ref[...].astype(jnp.bfloat16)
    xp = jnp.concatenate([x[:, 0 * _CONV_Q:1 * _CONV_Q],
                          x[:, 1 * _CONV_Q:2 * _CONV_Q],
                          x[:, 2 * _CONV_Q:3 * _CONV_Q],
                          x[:, 3 * _CONV_Q:4 * _CONV_Q]], axis=0)
    kk = lax.broadcasted_iota(jnp.int32, (4 * DIM, 4 * DIM), 0)
    cc = lax.broadcasted_iota(jnp.int32, (4 * DIM, 4 * DIM), 1)
    cm = cc & 127
    ktgt = ((cm >> 5) << 6) + ((cm & 31) << 1) + (cc >> 7)
    eye_p = (kk == ktgt).astype(jnp.bfloat16)
    z = jax.lax.dot_general(xp, eye_p, (((0,), (0,)), ((), ())),
                            preferred_element_type=jnp.float32)
    zi = jax.lax.bitcast_convert_type(z, jnp.int32)
    lo = jax.lax.shift_right_logical(zi[:, 0:PACKED_DIM], 16)
    out_ref[...] = zi[:, PACKED_DIM:2 * PACKED_DIM] | lo


_convert = pl.pallas_call(
    _conv_body,
    grid=(_CONV_GRID,),
    in_specs=[pl.BlockSpec((DIM, _CONV_ENTS), lambda j: (0, j))],
    out_specs=pl.BlockSpec((_CONV_Q, PACKED_DIM), lambda j: (j, 0)),
    out_shape=jax.ShapeDtypeStruct((_CONV_GRID * _CONV_Q, PACKED_DIM),
                                   jnp.int32),
)


def kernel(sample, entity_embedding, relation_embedding):
    idx = sample.astype(jnp.int32)
    packed = ((idx >> 15) << 13) | (idx & (_CONV_Q - 1))
    half = ((idx >> 13) & 3) << 5
    ent2 = _convert(entity_embedding.T)
    rel2 = _convert(relation_embedding.T)
    idx8 = jnp.stack([packed[:, 0], packed[:, 1], packed[:, 2],
                      half[:, 0], half[:, 1], half[:, 2],
                      half[:, 0], half[:, 0]])
    scores = _transe_sc(idx8, ent2, rel2)
    return scores.reshape(BATCH, 1)
